# baseline (speedup 1.0000x reference)
import functools

import jax
import jax.numpy as jnp
from jax import lax
from jax.experimental import pallas as pl
from jax.experimental.pallas import tpu as pltpu
from jax.experimental.pallas import tpu_sc as plsc

B = 4096
S = 200
D = 64
NC = 2
NS = 16
NW = NC * NS
ROWS = B * S
RPW = ROWS // NW
CB = 2
C = CB * S
NCH = RPW // C
SCALE = 8.0


def _mesh():
    return plsc.VectorSubcoreMesh(core_axis_name="c", subcore_axis_name="s")


@functools.partial(
    pl.kernel,
    mesh=_mesh(),
    out_type=jax.ShapeDtypeStruct((B, S, D), jnp.float32),
    compiler_params=pltpu.CompilerParams(use_tc_tiling_on_sc=False),
    scratch_types=[
        pltpu.VMEM((2, C), jnp.int32),
        pltpu.VMEM((2, C, D), jnp.float32),
        pltpu.VMEM((2, CB, S, D), jnp.float32),
        pltpu.VMEM((S, D), jnp.float32),
        pltpu.SemaphoreType.DMA,
        pltpu.SemaphoreType.DMA,
        pltpu.SemaphoreType.DMA,
        pltpu.SemaphoreType.DMA,
        pltpu.SemaphoreType.DMA,
        pltpu.SemaphoreType.DMA,
    ],
)
def _embed(idx_hbm, tok_hbm, pos_hbm, out_hbm, idx_v, g_v, o_v, pos_v,
           isem0, isem1, gsem0, gsem1, osem0, osem1):
    isem = (isem0, isem1)
    gsem = (gsem0, gsem1)
    osem = (osem0, osem1)
    wid = lax.axis_index("s") * NC + lax.axis_index("c")
    base = wid * RPW
    batch0 = wid * (RPW // S)
    pltpu.sync_copy(pos_hbm, pos_v)

    for b in range(2):
        pltpu.sync_copy(idx_hbm.at[pl.ds(base + b * C, C)], idx_v.at[b])
        pltpu.make_async_copy(tok_hbm.at[idx_v.at[b]], g_v.at[b],
                              gsem[b]).start()

    def out_copy(c, b):
        return pltpu.make_async_copy(
            o_v.at[b], out_hbm.at[pl.ds(batch0 + c * CB, CB)], osem[b])

    def outer(gi, carry):
        for b in range(2):
            c = 2 * gi + b
            row0 = base + c * C
            pltpu.make_async_copy(tok_hbm.at[idx_v.at[b]], g_v.at[b],
                                  gsem[b]).wait()
            @pl.when(c < NCH - 2)
            def _stage():
                pltpu.make_async_copy(
                    idx_hbm.at[pl.ds(row0 + 2 * C, C)], idx_v.at[b],
                    isem[b]).start()

            @pl.when(c >= 2)
            def _drain():
                out_copy(c, b).wait()

            def row_body(r, carry2):
                for j in range(D // 16):
                    sl = pl.ds(j * 16, 16)
                    pv = pos_v[r, sl]
                    for sb in range(CB):
                        o_v[b, sb, r, sl] = (
                            g_v[b, sb * S + r, sl] * SCALE + pv)
                return carry2

            lax.fori_loop(0, S, row_body, 0, unroll=2)

            out_copy(c, b).start()

            @pl.when(c < NCH - 2)
            def _next_gather():
                pltpu.make_async_copy(
                    idx_hbm.at[pl.ds(row0 + 2 * C, C)], idx_v.at[b],
                    isem[b]).wait()
                pltpu.make_async_copy(tok_hbm.at[idx_v.at[b]], g_v.at[b],
                                      gsem[b]).start()
        return carry

    lax.fori_loop(0, NCH // 2, outer, 0)
    for b in range(2):
        out_copy(NCH - 2 + b, b).wait()


def kernel(inputs, tok_table, pos_table):
    idx = inputs.reshape(ROWS).astype(jnp.int32)
    return _embed(idx, tok_table, pos_table)


# speedup vs baseline: 1.0572x; 1.0572x over previous
import functools

import jax
import jax.numpy as jnp
from jax import lax
from jax.experimental import pallas as pl
from jax.experimental.pallas import tpu as pltpu
from jax.experimental.pallas import tpu_sc as plsc

B = 4096
S = 200
D = 64
NC = 2
NS = 16
NW = NC * NS
ROWS = B * S
RPW = ROWS // NW
CB = 1
C = CB * S
NCH = RPW // C
SCALE = 8.0


def _mesh():
    return plsc.VectorSubcoreMesh(core_axis_name="c", subcore_axis_name="s")


@functools.partial(
    pl.kernel,
    mesh=_mesh(),
    out_type=jax.ShapeDtypeStruct((B, S, 2 * D), jnp.float32),
    compiler_params=pltpu.CompilerParams(use_tc_tiling_on_sc=False),
    scratch_types=[
        pltpu.VMEM((2, C), jnp.int32),
        pltpu.VMEM((2, C, D), jnp.float32),
        pltpu.VMEM((2, CB, S, 2 * D), jnp.float32),
        pltpu.VMEM((S, D), jnp.float32),
        pltpu.SemaphoreType.DMA,
        pltpu.SemaphoreType.DMA,
        pltpu.SemaphoreType.DMA,
        pltpu.SemaphoreType.DMA,
        pltpu.SemaphoreType.DMA,
        pltpu.SemaphoreType.DMA,
    ],
)
def _embed(idx_hbm, tok_hbm, pos_hbm, out_hbm, idx_v, g_v, o_v, pos_v,
           isem0, isem1, gsem0, gsem1, osem0, osem1):
    isem = (isem0, isem1)
    gsem = (gsem0, gsem1)
    osem = (osem0, osem1)
    wid = lax.axis_index("s") * NC + lax.axis_index("c")
    base = wid * RPW
    batch0 = wid * (RPW // S)
    pltpu.sync_copy(pos_hbm, pos_v)

    for b in range(2):
        pltpu.sync_copy(idx_hbm.at[pl.ds(base + b * C, C)], idx_v.at[b])
        pltpu.make_async_copy(tok_hbm.at[idx_v.at[b]], g_v.at[b],
                              gsem[b]).start()

    def out_copy(c, b):
        return pltpu.make_async_copy(
            o_v.at[b], out_hbm.at[pl.ds(batch0 + c * CB, CB)], osem[b])

    def outer(gi, carry):
        for b in range(2):
            c = 2 * gi + b
            row0 = base + c * C
            pltpu.make_async_copy(tok_hbm.at[idx_v.at[b]], g_v.at[b],
                                  gsem[b]).wait()
            @pl.when(c < NCH - 2)
            def _stage():
                pltpu.make_async_copy(
                    idx_hbm.at[pl.ds(row0 + 2 * C, C)], idx_v.at[b],
                    isem[b]).start()

            @pl.when(c >= 2)
            def _drain():
                out_copy(c, b).wait()

            def row_body(r, carry2):
                for j in range(D // 16):
                    sl = pl.ds(j * 16, 16)
                    pv = pos_v[r, sl]
                    for sb in range(CB):
                        o_v[b, sb, r, sl] = (
                            g_v[b, sb * S + r, sl] * SCALE + pv)
                return carry2

            lax.fori_loop(0, S, row_body, 0, unroll=2)

            out_copy(c, b).start()

            @pl.when(c < NCH - 2)
            def _next_gather():
                pltpu.make_async_copy(
                    idx_hbm.at[pl.ds(row0 + 2 * C, C)], idx_v.at[b],
                    isem[b]).wait()
                pltpu.make_async_copy(tok_hbm.at[idx_v.at[b]], g_v.at[b],
                                      gsem[b]).start()
        return carry

    lax.fori_loop(0, NCH // 2, outer, 0)
    for b in range(2):
        out_copy(NCH - 2 + b, b).wait()


def kernel(inputs, tok_table, pos_table):
    idx = inputs.reshape(ROWS).astype(jnp.int32)
    return _embed(idx, tok_table, pos_table)[:, :, :D]


# speedup vs baseline: 1.2685x; 1.1999x over previous
import functools

import jax
import jax.numpy as jnp
from jax import lax
from jax.experimental import pallas as pl
from jax.experimental.pallas import tpu as pltpu
from jax.experimental.pallas import tpu_sc as plsc

B = 4096
S = 200
D = 64
NC = 2
NS = 16
NW = NC * NS
ROWS = B * S
RPW = ROWS // NW
CB = 2
C = CB * S
NCH = RPW // C
SCALE = 8.0


def _mesh():
    return plsc.VectorSubcoreMesh(core_axis_name="c", subcore_axis_name="s")


@functools.partial(
    pl.kernel,
    mesh=_mesh(),
    out_type=jax.ShapeDtypeStruct((B, S, 2 * D), jnp.float32),
    compiler_params=pltpu.CompilerParams(use_tc_tiling_on_sc=False),
    scratch_types=[
        pltpu.VMEM((2, C), jnp.int32),
        pltpu.VMEM((2, C, D), jnp.float32),
        pltpu.VMEM((2, CB, S, D), jnp.float32),
        pltpu.VMEM((S, D), jnp.float32),
        pltpu.SemaphoreType.DMA,
        pltpu.SemaphoreType.DMA,
        pltpu.SemaphoreType.DMA,
        pltpu.SemaphoreType.DMA,
        pltpu.SemaphoreType.DMA,
        pltpu.SemaphoreType.DMA,
    ],
)
def _embed(idx_hbm, tok_hbm, pos_hbm, out_hbm, idx_v, g_v, o_v, pos_v,
           isem0, isem1, gsem0, gsem1, osem0, osem1):
    isem = (isem0, isem1)
    gsem = (gsem0, gsem1)
    osem = (osem0, osem1)
    wid = lax.axis_index("s") * NC + lax.axis_index("c")
    base = wid * RPW
    batch0 = wid * (RPW // S)
    pltpu.sync_copy(pos_hbm, pos_v)

    for b in range(2):
        pltpu.sync_copy(idx_hbm.at[pl.ds(base + b * C, C)], idx_v.at[b])
        pltpu.make_async_copy(tok_hbm.at[idx_v.at[b]], g_v.at[b],
                              gsem[b]).start()

    def out_copy(c, b):
        return pltpu.make_async_copy(
            o_v.at[b],
            out_hbm.at[pl.ds(batch0 + c * CB, CB), pl.ds(0, S), pl.ds(0, D)],
            osem[b])

    def outer(gi, carry):
        for b in range(2):
            c = 2 * gi + b
            row0 = base + c * C
            pltpu.make_async_copy(tok_hbm.at[idx_v.at[b]], g_v.at[b],
                                  gsem[b]).wait()
            @pl.when(c < NCH - 2)
            def _stage():
                pltpu.make_async_copy(
                    idx_hbm.at[pl.ds(row0 + 2 * C, C)], idx_v.at[b],
                    isem[b]).start()

            @pl.when(c >= 2)
            def _drain():
                out_copy(c, b).wait()

            def row_body(r, carry2):
                for j in range(D // 16):
                    sl = pl.ds(j * 16, 16)
                    pv = pos_v[r, sl]
                    for sb in range(CB):
                        o_v[b, sb, r, sl] = (
                            g_v[b, sb * S + r, sl] * SCALE + pv)
                return carry2

            lax.fori_loop(0, S, row_body, 0, unroll=2)

            out_copy(c, b).start()

            @pl.when(c < NCH - 2)
            def _next_gather():
                pltpu.make_async_copy(
                    idx_hbm.at[pl.ds(row0 + 2 * C, C)], idx_v.at[b],
                    isem[b]).wait()
                pltpu.make_async_copy(tok_hbm.at[idx_v.at[b]], g_v.at[b],
                                      gsem[b]).start()
        return carry

    lax.fori_loop(0, NCH // 2, outer, 0)
    for b in range(2):
        out_copy(NCH - 2 + b, b).wait()


def kernel(inputs, tok_table, pos_table):
    idx = inputs.reshape(ROWS).astype(jnp.int32)
    return _embed(idx, tok_table, pos_table)[:, :, :D]


# speedup vs baseline: 1.2710x; 1.0020x over previous
import functools

import jax
import jax.numpy as jnp
from jax import lax
from jax.experimental import pallas as pl
from jax.experimental.pallas import tpu as pltpu
from jax.experimental.pallas import tpu_sc as plsc

B = 4096
S = 200
D = 64
NC = 2
NS = 16
NW = NC * NS
ROWS = B * S
RPW = ROWS // NW
CB = 2
C = CB * S
NCH = RPW // C
SCALE = 8.0


def _mesh():
    return plsc.VectorSubcoreMesh(core_axis_name="c", subcore_axis_name="s")


@functools.partial(
    pl.kernel,
    mesh=_mesh(),
    out_type=jax.ShapeDtypeStruct((B, S, 2 * D), jnp.float32),
    compiler_params=pltpu.CompilerParams(use_tc_tiling_on_sc=False),
    scratch_types=[
        pltpu.VMEM((2, C), jnp.int32),
        pltpu.VMEM((2, C, D), jnp.float32),
        pltpu.VMEM((2, CB, S, D), jnp.float32),
        pltpu.VMEM((S, D), jnp.float32),
        pltpu.SemaphoreType.DMA,
        pltpu.SemaphoreType.DMA,
        pltpu.SemaphoreType.DMA,
        pltpu.SemaphoreType.DMA,
        pltpu.SemaphoreType.DMA,
        pltpu.SemaphoreType.DMA,
    ],
)
def _embed(idx_hbm, tok_hbm, pos_hbm, out_hbm, idx_v, g_v, o_v, pos_v,
           isem0, isem1, gsem0, gsem1, osem0, osem1):
    isem = (isem0, isem1)
    gsem = (gsem0, gsem1)
    osem = (osem0, osem1)
    wid = lax.axis_index("s") * NC + lax.axis_index("c")
    base = wid * RPW
    batch0 = wid * (RPW // S)
    pltpu.sync_copy(pos_hbm, pos_v)

    H = C // 2

    def gathers(b):
        return [
            pltpu.make_async_copy(
                tok_hbm.at[idx_v.at[b, pl.ds(h * H, H)]],
                g_v.at[b, pl.ds(h * H, H)], gsem[b])
            for h in range(2)
        ]

    for b in range(2):
        pltpu.sync_copy(idx_hbm.at[pl.ds(base + b * C, C)], idx_v.at[b])
        for g in gathers(b):
            g.start()

    def out_copy(c, b):
        return pltpu.make_async_copy(
            o_v.at[b],
            out_hbm.at[pl.ds(batch0 + c * CB, CB), pl.ds(0, S), pl.ds(0, D)],
            osem[b])

    def outer(gi, carry):
        for b in range(2):
            c = 2 * gi + b
            row0 = base + c * C
            for g in gathers(b):
                g.wait()
            @pl.when(c < NCH - 2)
            def _stage():
                pltpu.make_async_copy(
                    idx_hbm.at[pl.ds(row0 + 2 * C, C)], idx_v.at[b],
                    isem[b]).start()

            @pl.when(c >= 2)
            def _drain():
                out_copy(c, b).wait()

            def row_body(r, carry2):
                for j in range(D // 16):
                    sl = pl.ds(j * 16, 16)
                    pv = pos_v[r, sl]
                    for sb in range(CB):
                        o_v[b, sb, r, sl] = (
                            g_v[b, sb * S + r, sl] * SCALE + pv)
                return carry2

            lax.fori_loop(0, S, row_body, 0, unroll=2)

            out_copy(c, b).start()

            @pl.when(c < NCH - 2)
            def _next_gather():
                pltpu.make_async_copy(
                    idx_hbm.at[pl.ds(row0 + 2 * C, C)], idx_v.at[b],
                    isem[b]).wait()
                for g in gathers(b):
                    g.start()
        return carry

    lax.fori_loop(0, NCH // 2, outer, 0)
    for b in range(2):
        out_copy(NCH - 2 + b, b).wait()


def kernel(inputs, tok_table, pos_table):
    idx = inputs.reshape(ROWS).astype(jnp.int32)
    return _embed(idx, tok_table, pos_table)[:, :, :D]
